# baseline (speedup 1.0000x reference)
import functools

import jax
import jax.numpy as jnp
from jax import lax
from jax.experimental import pallas as pl
from jax.experimental.pallas import tpu as pltpu
from jax.experimental.pallas import tpu_sc as plsc

B, S, D = 4, 4096, 2048
NC, NS, L = 2, 16, 16
NW = NC * NS
SEQ_PER_W = S // NW
R = 16
CHUNKS = SEQ_PER_W // R
CH_W = R * D
UNITS = CHUNKS * B


def _sc_body(x_hbm, pos_hbm, out_hbm, pos_v, xa, xb, sem_p, sem_la, sem_lb,
             sem_sa, sem_sb):
    wid = lax.axis_index("s") * NC + lax.axis_index("c")
    base = wid * SEQ_PER_W * D

    xbuf = (xa, xb)
    lsem = (sem_la, sem_lb)

    def x_off(u):
        c, b = divmod(u, B)
        return b * (S * D) + base + c * CH_W

    pos_d = pltpu.async_copy(pos_hbm.at[pl.ds(base, CH_W)], pos_v, sem_p)
    load_d = [None] * UNITS
    store_d = [None] * UNITS
    load_d[0] = pltpu.async_copy(x_hbm.at[pl.ds(x_off(0), CH_W)], xa, sem_la)

    for u in range(UNITS):
        c, b = divmod(u, B)
        cur = xbuf[u % 2]
        if u + 1 < UNITS:
            if u >= 1:
                store_d[u - 1].wait()
            nxt = xbuf[(u + 1) % 2]
            load_d[u + 1] = pltpu.async_copy(
                x_hbm.at[pl.ds(x_off(u + 1), CH_W)], nxt, lsem[(u + 1) % 2])
        if b == 0:
            pos_d.wait()
        load_d[u].wait()

        @plsc.parallel_loop(0, CH_W // L, unroll=16)
        def add_body(i):
            sl = pl.ds(i * L, L)
            plsc.addupdate(cur.at[sl], pos_v[sl])

        ssem = sem_sa if u % 2 == 0 else sem_sb
        store_d[u] = pltpu.async_copy(cur, out_hbm.at[pl.ds(x_off(u), CH_W)],
                                      ssem)
        if b == B - 1 and c + 1 < CHUNKS:
            pos_d = pltpu.async_copy(
                pos_hbm.at[pl.ds(base + (c + 1) * CH_W, CH_W)], pos_v, sem_p)

    store_d[UNITS - 2].wait()
    store_d[UNITS - 1].wait()


@jax.jit
def _sc_add(xf, pf):
    mesh = plsc.VectorSubcoreMesh(core_axis_name="c", subcore_axis_name="s")
    return pl.kernel(
        _sc_body,
        mesh=mesh,
        out_type=jax.ShapeDtypeStruct((B * S * D,), jnp.float32),
        scratch_types=[
            pltpu.VMEM((CH_W,), jnp.float32),
            pltpu.VMEM((CH_W,), jnp.float32),
            pltpu.VMEM((CH_W,), jnp.float32),
            pltpu.SemaphoreType.DMA,
            pltpu.SemaphoreType.DMA,
            pltpu.SemaphoreType.DMA,
            pltpu.SemaphoreType.DMA,
            pltpu.SemaphoreType.DMA,
        ],
    )(xf, pf)


def kernel(x, pos_table):
    xf = x.reshape(-1)
    pf = pos_table.reshape(-1)
    out = _sc_add(xf, pf)
    return out.reshape(x.shape)


# speedup vs baseline: 1.0564x; 1.0564x over previous
import functools

import jax
import jax.numpy as jnp
from jax import lax
from jax.experimental import pallas as pl
from jax.experimental.pallas import tpu as pltpu
from jax.experimental.pallas import tpu_sc as plsc

B, S, D = 4, 4096, 2048
NC, NS, L = 2, 16, 16
NW = NC * NS
SEQ_PER_W = S // NW
R = 8
CHUNKS = SEQ_PER_W // R
CH_W = R * D
UNITS = CHUNKS * B
NBUF = 4
DEPTH = 2


def _sc_body(x_hbm, pos_hbm, out_hbm,
             x0, x1, x2, x3, p0, p1,
             sl0, sl1, sl2, sl3, ss0, ss1, ss2, ss3, sp0, sp1):
    wid = lax.axis_index("s") * NC + lax.axis_index("c")
    base = wid * SEQ_PER_W * D

    xbuf = (x0, x1, x2, x3)
    lsem = (sl0, sl1, sl2, sl3)
    ssem = (ss0, ss1, ss2, ss3)
    pbuf = (p0, p1)
    psem = (sp0, sp1)

    def x_off(u):
        c, b = divmod(u, B)
        return b * (S * D) + base + c * CH_W

    def start_load(u):
        return pltpu.async_copy(
            x_hbm.at[pl.ds(x_off(u), CH_W)], xbuf[u % NBUF], lsem[u % NBUF])

    def start_pos(c):
        return pltpu.async_copy(
            pos_hbm.at[pl.ds(base + c * CH_W, CH_W)], pbuf[c % 2], psem[c % 2])

    load_d = [None] * UNITS
    store_d = [None] * UNITS
    pos_d = [None] * CHUNKS

    pos_d[0] = start_pos(0)
    for u in range(DEPTH):
        load_d[u] = start_load(u)

    for u in range(UNITS):
        c, b = divmod(u, B)
        cur = xbuf[u % NBUF]
        pos_v = pbuf[c % 2]
        if u + DEPTH < UNITS:
            prev = u + DEPTH - NBUF
            if prev >= 0:
                store_d[prev].wait()
            load_d[u + DEPTH] = start_load(u + DEPTH)
        if b == 0:
            if c + 1 < CHUNKS:
                pos_d[c + 1] = start_pos(c + 1)
            pos_d[c].wait()
        load_d[u].wait()

        @plsc.parallel_loop(0, CH_W // L, unroll=16)
        def add_body(i):
            sl = pl.ds(i * L, L)
            plsc.addupdate(cur.at[sl], pos_v[sl])

        store_d[u] = pltpu.async_copy(cur, out_hbm.at[pl.ds(x_off(u), CH_W)],
                                      ssem[u % NBUF])

    for u in range(UNITS - NBUF, UNITS):
        store_d[u].wait()


@jax.jit
def _sc_add(xf, pf):
    mesh = plsc.VectorSubcoreMesh(core_axis_name="c", subcore_axis_name="s")
    return pl.kernel(
        _sc_body,
        mesh=mesh,
        out_type=jax.ShapeDtypeStruct((B * S * D,), jnp.float32),
        scratch_types=(
            [pltpu.VMEM((CH_W,), jnp.float32)] * (NBUF + 2)
            + [pltpu.SemaphoreType.DMA] * (NBUF * 2 + 2)
        ),
    )(xf, pf)


def kernel(x, pos_table):
    xf = x.reshape(-1)
    pf = pos_table.reshape(-1)
    out = _sc_add(xf, pf)
    return out.reshape(x.shape)


# speedup vs baseline: 1.5582x; 1.4749x over previous
import functools

import jax
import jax.numpy as jnp
from jax import lax
from jax.experimental import pallas as pl
from jax.experimental.pallas import tpu as pltpu
from jax.experimental.pallas import tpu_sc as plsc

B, S, D = 4, 4096, 2048
NC, NS, L = 2, 16, 16
NW = NC * NS
SEQ_PER_W = S // NW
R = 8
CHUNKS = SEQ_PER_W // R
CH_W = R * D
NBUF = 4
DEPTH = 2

TC_B = B - 1
BS = 512


def _sc_body(x_hbm, pos_hbm, out_hbm,
             x0, x1, x2, x3, p0, p1,
             sl0, sl1, sl2, sl3, ss0, ss1, ss2, ss3, sp0, sp1):
    wid = lax.axis_index("s") * NC + lax.axis_index("c")
    base = wid * SEQ_PER_W * D

    xbuf = (x0, x1, x2, x3)
    lsem = (sl0, sl1, sl2, sl3)
    ssem = (ss0, ss1, ss2, ss3)
    pbuf = (p0, p1)
    psem = (sp0, sp1)

    def start_load(u):
        return pltpu.async_copy(
            x_hbm.at[pl.ds(base + u * CH_W, CH_W)], xbuf[u % NBUF],
            lsem[u % NBUF])

    def start_pos(u):
        return pltpu.async_copy(
            pos_hbm.at[pl.ds(base + u * CH_W, CH_W)], pbuf[u % 2],
            psem[u % 2])

    load_d = [None] * CHUNKS
    store_d = [None] * CHUNKS
    pos_d = [None] * CHUNKS

    for u in range(DEPTH):
        load_d[u] = start_load(u)
    pos_d[0] = start_pos(0)

    for u in range(CHUNKS):
        cur = xbuf[u % NBUF]
        pos_v = pbuf[u % 2]
        if u + DEPTH < CHUNKS:
            prev = u + DEPTH - NBUF
            if prev >= 0:
                store_d[prev].wait()
            load_d[u + DEPTH] = start_load(u + DEPTH)
        pos_d[u].wait()
        if u + 1 < CHUNKS:
            pos_d[u + 1] = start_pos(u + 1)
        load_d[u].wait()

        @plsc.parallel_loop(0, CH_W // L, unroll=16)
        def add_body(i):
            sl = pl.ds(i * L, L)
            plsc.addupdate(cur.at[sl], pos_v[sl])

        store_d[u] = pltpu.async_copy(
            cur, out_hbm.at[pl.ds(base + u * CH_W, CH_W)], ssem[u % NBUF])

    for u in range(CHUNKS - NBUF, CHUNKS):
        store_d[u].wait()


@jax.jit
def _hybrid(x, pos_table):
    mesh = plsc.VectorSubcoreMesh(core_axis_name="c", subcore_axis_name="s")
    sc_out = pl.kernel(
        _sc_body,
        mesh=mesh,
        out_type=jax.ShapeDtypeStruct((S * D,), jnp.float32),
        scratch_types=(
            [pltpu.VMEM((CH_W,), jnp.float32)] * (NBUF + 2)
            + [pltpu.SemaphoreType.DMA] * (NBUF * 2 + 2)
        ),
    )(x[0].reshape(-1), pos_table.reshape(-1))

    def tc_body(x_ref, p_ref, o_ref):
        o_ref[...] = x_ref[...] + p_ref[...]

    tc_full = pl.pallas_call(
        tc_body,
        grid=(S // BS, TC_B),
        in_specs=[
            pl.BlockSpec((1, BS, D), lambda i, b: (b + 1, i, 0)),
            pl.BlockSpec((BS, D), lambda i, b: (i, 0)),
        ],
        out_specs=pl.BlockSpec((1, BS, D), lambda i, b: (b + 1, i, 0)),
        out_shape=jax.ShapeDtypeStruct((B, S, D), jnp.float32),
    )(x, pos_table)

    return lax.dynamic_update_slice(tc_full, sc_out.reshape(1, S, D),
                                    (0, 0, 0))


def kernel(x, pos_table):
    return _hybrid(x, pos_table)
